# baseline (speedup 1.0000x reference)
import functools

import jax
import jax.numpy as jnp
from jax import lax
from jax.experimental import pallas as pl
from jax.experimental.pallas import tpu as pltpu
from jax.experimental.pallas import tpu_sc as plsc

N = 10000
NE = 320000
D_IN = 128
D_HID = 128
D_OUT = 40

NW = 32
EPT = NE // NW
K = 80
NCH = EPT // K

RPW = 640
TAIL = N - 15 * RPW

_mesh = plsc.VectorSubcoreMesh(core_axis_name="c", subcore_axis_name="s")


def _zero_acc(zeros_hbm, acc, s):
    @pl.when(s < 15)
    def _():
        pltpu.sync_copy(zeros_hbm, acc.at[pl.ds(s * RPW, RPW)])

    @pl.when(s == 15)
    def _():
        pltpu.sync_copy(zeros_hbm.at[pl.ds(0, TAIL)], acc.at[pl.ds(15 * RPW, TAIL)])


def _write_out(acc, out_hbm, c, s):
    @pl.when(s < 15)
    def _():
        pltpu.sync_copy(acc.at[pl.ds(s * RPW, RPW)], out_hbm.at[c, pl.ds(s * RPW, RPW)])

    @pl.when(s == 15)
    def _():
        pltpu.sync_copy(acc.at[pl.ds(15 * RPW, TAIL)], out_hbm.at[c, pl.ds(15 * RPW, TAIL)])


DEGW = 128


def _make_deg_kernel():
    @functools.partial(
        pl.kernel,
        out_type=jax.ShapeDtypeStruct((2, N, DEGW), jnp.float32),
        mesh=_mesh,
        scratch_types=[
            pltpu.VMEM((NCH, K), jnp.int32),
            pltpu.VMEM((K, DEGW), jnp.float32),
            pltpu.VMEM_SHARED((N, DEGW), jnp.float32),
        ],
        compiler_params=pltpu.CompilerParams(use_tc_tiling_on_sc=False),
    )
    def deg_kernel(dsts_hbm, ones_hbm, zeros_hbm, out_hbm, dst_v, ones_v, acc):
        c = lax.axis_index("c")
        s = lax.axis_index("s")
        w = c * 16 + s
        pltpu.sync_copy(dsts_hbm.at[w], dst_v)
        pltpu.sync_copy(ones_hbm, ones_v)
        _zero_acc(zeros_hbm, acc, s)
        plsc.subcore_barrier()

        def body(j, carry):
            pltpu.sync_copy(ones_v, acc.at[dst_v.at[j]], add=True)
            return carry

        lax.fori_loop(0, NCH, body, 0)
        plsc.subcore_barrier()
        _write_out(acc, out_hbm, c, s)

    return deg_kernel


def _make_agg_kernel(D):
    @functools.partial(
        pl.kernel,
        out_type=jax.ShapeDtypeStruct((2, N, D), jnp.float32),
        mesh=_mesh,
        scratch_types=[
            pltpu.VMEM((3, K), jnp.int32),
            pltpu.VMEM((4, K), jnp.int32),
            pltpu.VMEM((3, K, D), jnp.float32),
            pltpu.VMEM_SHARED((N, D), jnp.float32),
            pltpu.SemaphoreType.DMA,
            pltpu.SemaphoreType.DMA,
            pltpu.SemaphoreType.DMA,
        ],
        compiler_params=pltpu.CompilerParams(use_tc_tiling_on_sc=False),
    )
    def agg_kernel(hs_hbm, srcs_hbm, dsts_hbm, zeros_hbm, out_hbm,
                   src_v, dst_v, stage, acc, gsem, isem, ssem):
        c = lax.axis_index("c")
        s = lax.axis_index("s")
        w = c * 16 + s
        _zero_acc(zeros_hbm, acc, s)

        def issue_gather(p3):
            pltpu.async_copy(hs_hbm.at[src_v.at[p3]], stage.at[p3], gsem)

        def wait_gather(p3):
            pltpu.make_async_copy(hs_hbm.at[src_v.at[p3]], stage.at[p3], gsem).wait()

        def issue_scatter(p3, p4):
            pltpu.async_copy(stage.at[p3], acc.at[dst_v.at[p4]], ssem, add=True)

        def wait_scatter(p3, p4):
            pltpu.make_async_copy(stage.at[p3], acc.at[dst_v.at[p4]], ssem).wait()

        def prefetch_idx(j, s3, d4):
            pltpu.async_copy(srcs_hbm.at[w, j], src_v.at[s3], isem)
            pltpu.async_copy(dsts_hbm.at[w, j], dst_v.at[d4], isem)

        def wait_idx(s3, d4):
            pltpu.make_async_copy(srcs_hbm.at[w, 0], src_v.at[s3], isem).wait()
            pltpu.make_async_copy(dsts_hbm.at[w, 0], dst_v.at[d4], isem).wait()

        def step(j, p3, pn3, pf3, p4, pn4, pf4, w3, w4,
                 do_swait, do_next, do_pref):
            wait_gather(p3)
            if do_swait:
                wait_scatter(w3, w4)
            issue_scatter(p3, p4)
            if do_next:
                wait_idx(pn3, pn4)
                issue_gather(pn3)
            if do_pref:
                prefetch_idx(j + 2, pf3, pf4)

        pltpu.sync_copy(srcs_hbm.at[w, 0], src_v.at[0])
        pltpu.sync_copy(dsts_hbm.at[w, 0], dst_v.at[0])
        plsc.subcore_barrier()
        issue_gather(0)
        prefetch_idx(1, 1, 1)

        step(0, 0, 1, 2, 0, 1, 2, 0, 0, False, True, True)
        step(1, 1, 2, 0, 1, 2, 3, 0, 0, False, True, True)

        def body(j, carry):
            p3 = lax.rem(j, 3)
            pn3 = lax.rem(j + 1, 3)
            pf3 = lax.rem(j + 2, 3)
            p4 = lax.rem(j, 4)
            pn4 = lax.rem(j + 1, 4)
            pf4 = lax.rem(j + 2, 4)
            step(j, p3, pn3, pf3, p4, pn4, pf4, pn3, pf4, True, True, True)
            return carry

        lax.fori_loop(2, NCH - 2, body, 0)

        jA = NCH - 2
        jB = NCH - 1
        step(jA, jA % 3, jB % 3, 0, jA % 4, jB % 4, 0,
             (jA - 2) % 3, (jA - 2) % 4, True, True, False)
        step(jB, jB % 3, 0, 0, jB % 4, 0, 0,
             (jB - 2) % 3, (jB - 2) % 4, True, False, False)

        wait_scatter(jA % 3, jA % 4)
        wait_scatter(jB % 3, jB % 4)

        plsc.subcore_barrier()
        _write_out(acc, out_hbm, c, s)

    return agg_kernel


_deg_kernel = _make_deg_kernel()
_agg128 = _make_agg_kernel(D_HID)
_agg40 = _make_agg_kernel(D_OUT)

_TCB = 1000


def _tc1_body(deg_ref, x_ref, w_ref, hs_ref, dinv_ref):
    deg = deg_ref[0, :, 0:1] + deg_ref[1, :, 0:1] + 1.0
    dinv = lax.rsqrt(deg)
    h = jnp.dot(x_ref[...], w_ref[...], preferred_element_type=jnp.float32)
    hs_ref[...] = h * dinv
    dinv_ref[...] = dinv


def _tc1(degp, x, W1):
    grid = (N // _TCB,)
    return pl.pallas_call(
        _tc1_body,
        grid=grid,
        in_specs=[
            pl.BlockSpec((2, _TCB, DEGW), lambda i: (0, i, 0)),
            pl.BlockSpec((_TCB, D_IN), lambda i: (i, 0)),
            pl.BlockSpec((D_IN, D_HID), lambda i: (0, 0)),
        ],
        out_specs=[
            pl.BlockSpec((_TCB, D_HID), lambda i: (i, 0)),
            pl.BlockSpec((_TCB, 1), lambda i: (i, 0)),
        ],
        out_shape=[
            jax.ShapeDtypeStruct((N, D_HID), jnp.float32),
            jax.ShapeDtypeStruct((N, 1), jnp.float32),
        ],
    )(degp, x, W1)


def _tc2_body(agg_ref, hs1_ref, dinv_ref, b1_ref, w2_ref, hs2_ref):
    dinv = dinv_ref[...]
    o = (agg_ref[0] + agg_ref[1] + hs1_ref[...]) * dinv + b1_ref[...]
    o = jnp.maximum(o, 0.0)
    h2 = jnp.dot(o, w2_ref[...], preferred_element_type=jnp.float32)
    hs2_ref[...] = h2 * dinv


def _tc2(agg, hs1, dinv, b1, W2):
    grid = (N // _TCB,)
    return pl.pallas_call(
        _tc2_body,
        grid=grid,
        in_specs=[
            pl.BlockSpec((2, _TCB, D_HID), lambda i: (0, i, 0)),
            pl.BlockSpec((_TCB, D_HID), lambda i: (i, 0)),
            pl.BlockSpec((_TCB, 1), lambda i: (i, 0)),
            pl.BlockSpec((1, D_HID), lambda i: (0, 0)),
            pl.BlockSpec((D_HID, D_OUT), lambda i: (0, 0)),
        ],
        out_specs=pl.BlockSpec((_TCB, D_OUT), lambda i: (i, 0)),
        out_shape=jax.ShapeDtypeStruct((N, D_OUT), jnp.float32),
    )(agg, hs1, dinv, b1, W2)


def _tc3_body(agg_ref, hs2_ref, dinv_ref, b2_ref, out_ref):
    z = (agg_ref[0] + agg_ref[1] + hs2_ref[...]) * dinv_ref[...] + b2_ref[...]
    m = jnp.max(z, axis=1, keepdims=True)
    e = jnp.exp(z - m)
    lse = jnp.log(jnp.sum(e, axis=1, keepdims=True)) + m
    out_ref[...] = z - lse


def _tc3(agg, hs2, dinv, b2):
    grid = (N // _TCB,)
    return pl.pallas_call(
        _tc3_body,
        grid=grid,
        in_specs=[
            pl.BlockSpec((2, _TCB, D_OUT), lambda i: (0, i, 0)),
            pl.BlockSpec((_TCB, D_OUT), lambda i: (i, 0)),
            pl.BlockSpec((_TCB, 1), lambda i: (i, 0)),
            pl.BlockSpec((1, D_OUT), lambda i: (0, 0)),
        ],
        out_specs=pl.BlockSpec((_TCB, D_OUT), lambda i: (i, 0)),
        out_shape=jax.ShapeDtypeStruct((N, D_OUT), jnp.float32),
    )(agg, hs2, dinv, b2)


def kernel(x, edge_index, W1, b1, W2, b2):
    e = edge_index.astype(jnp.int32)
    src_r = e[0].reshape(NW, NCH, K)
    dst_r = e[1].reshape(NW, NCH, K)

    ones_c = jnp.ones((K, DEGW), jnp.float32)
    zeros_c = jnp.zeros((RPW, DEGW), jnp.float32)
    zeros_h = jnp.zeros((RPW, D_HID), jnp.float32)
    zeros_o = jnp.zeros((RPW, D_OUT), jnp.float32)

    degp = _deg_kernel(dst_r, ones_c, zeros_c)
    hs1, dinv = _tc1(degp, x, W1)
    agg1 = _agg128(hs1, src_r, dst_r, zeros_h)
    hs2 = _tc2(agg1, hs1, dinv, b1.reshape(1, D_HID), W2)
    agg2 = _agg40(hs2, src_r, dst_r, zeros_o)
    return _tc3(agg2, hs2, dinv, b2.reshape(1, D_OUT))


# speedup vs baseline: 1.0673x; 1.0673x over previous
import functools

import jax
import jax.numpy as jnp
from jax import lax
from jax.experimental import pallas as pl
from jax.experimental.pallas import tpu as pltpu
from jax.experimental.pallas import tpu_sc as plsc

N = 10000
NE = 320000
D_IN = 128
D_HID = 128
D_OUT = 40

NW = 32
EPT = NE // NW
K = 80
NCH = EPT // K

RPW = 640
TAIL = N - 15 * RPW

_mesh = plsc.VectorSubcoreMesh(core_axis_name="c", subcore_axis_name="s")


def _zero_acc(zeros_hbm, acc, s):
    @pl.when(s < 15)
    def _():
        pltpu.sync_copy(zeros_hbm, acc.at[pl.ds(s * RPW, RPW)])

    @pl.when(s == 15)
    def _():
        pltpu.sync_copy(zeros_hbm.at[pl.ds(0, TAIL)], acc.at[pl.ds(15 * RPW, TAIL)])


def _write_out(acc, out_hbm, c, s):
    @pl.when(s < 15)
    def _():
        pltpu.sync_copy(acc.at[pl.ds(s * RPW, RPW)], out_hbm.at[c, pl.ds(s * RPW, RPW)])

    @pl.when(s == 15)
    def _():
        pltpu.sync_copy(acc.at[pl.ds(15 * RPW, TAIL)], out_hbm.at[c, pl.ds(15 * RPW, TAIL)])


DW = 16


def _make_deg_kernel():
    @functools.partial(
        pl.kernel,
        out_type=jax.ShapeDtypeStruct((2, N, DW), jnp.float32),
        mesh=_mesh,
        scratch_types=[
            pltpu.VMEM((4, K), jnp.int32),
            pltpu.VMEM((K, DW), jnp.float32),
            pltpu.VMEM_SHARED((N, DW), jnp.float32),
            pltpu.SemaphoreType.DMA,
            pltpu.SemaphoreType.DMA,
        ],
        compiler_params=pltpu.CompilerParams(use_tc_tiling_on_sc=False),
    )
    def deg_kernel(dsts_hbm, ones_hbm, zeros_hbm, out_hbm,
                   dst_v, ones_v, acc, isem, ssem):
        c = lax.axis_index("c")
        s = lax.axis_index("s")
        w = c * 16 + s
        _zero_acc(zeros_hbm, acc, s)
        pltpu.sync_copy(ones_hbm, ones_v)

        def issue_scatter(p4):
            pltpu.async_copy(ones_v, acc.at[dst_v.at[p4]], ssem, add=True)

        def wait_scatter(p4):
            pltpu.make_async_copy(ones_v, acc.at[dst_v.at[p4]], ssem).wait()

        def prefetch_idx(j, d4):
            pltpu.async_copy(dsts_hbm.at[w, j], dst_v.at[d4], isem)

        def wait_idx(d4):
            pltpu.make_async_copy(dsts_hbm.at[w, 0], dst_v.at[d4], isem).wait()

        pltpu.sync_copy(dsts_hbm.at[w, 0], dst_v.at[0])
        plsc.subcore_barrier()
        issue_scatter(0)
        prefetch_idx(1, 1)
        prefetch_idx(2, 2)
        wait_idx(1)
        issue_scatter(1)
        prefetch_idx(3, 3)

        def body(j, carry):
            p4 = lax.rem(j, 4)
            pf4 = lax.rem(j + 2, 4)
            w4 = lax.rem(j + 2, 4)
            wait_scatter(w4)
            wait_idx(p4)
            issue_scatter(p4)
            prefetch_idx(j + 2, pf4)
            return carry

        lax.fori_loop(2, NCH - 2, body, 0)

        jA = NCH - 2
        jB = NCH - 1
        wait_scatter((jA - 2) % 4)
        wait_idx(jA % 4)
        issue_scatter(jA % 4)
        wait_scatter((jB - 2) % 4)
        wait_idx(jB % 4)
        issue_scatter(jB % 4)
        wait_scatter(jA % 4)
        wait_scatter(jB % 4)

        plsc.subcore_barrier()
        _write_out(acc, out_hbm, c, s)

    return deg_kernel


def _make_agg_kernel(D):
    @functools.partial(
        pl.kernel,
        out_type=jax.ShapeDtypeStruct((2, N, D), jnp.float32),
        mesh=_mesh,
        scratch_types=[
            pltpu.VMEM((3, K), jnp.int32),
            pltpu.VMEM((4, K), jnp.int32),
            pltpu.VMEM((3, K, D), jnp.float32),
            pltpu.VMEM_SHARED((N, D), jnp.float32),
            pltpu.SemaphoreType.DMA,
            pltpu.SemaphoreType.DMA,
            pltpu.SemaphoreType.DMA,
        ],
        compiler_params=pltpu.CompilerParams(use_tc_tiling_on_sc=False),
    )
    def agg_kernel(hs_hbm, srcs_hbm, dsts_hbm, zeros_hbm, out_hbm,
                   src_v, dst_v, stage, acc, gsem, isem, ssem):
        c = lax.axis_index("c")
        s = lax.axis_index("s")
        w = c * 16 + s
        _zero_acc(zeros_hbm, acc, s)

        def issue_gather(p3):
            pltpu.async_copy(hs_hbm.at[src_v.at[p3]], stage.at[p3], gsem)

        def wait_gather(p3):
            pltpu.make_async_copy(hs_hbm.at[src_v.at[p3]], stage.at[p3], gsem).wait()

        def issue_scatter(p3, p4):
            pltpu.async_copy(stage.at[p3], acc.at[dst_v.at[p4]], ssem, add=True)

        def wait_scatter(p3, p4):
            pltpu.make_async_copy(stage.at[p3], acc.at[dst_v.at[p4]], ssem).wait()

        def prefetch_idx(j, s3, d4):
            pltpu.async_copy(srcs_hbm.at[w, j], src_v.at[s3], isem)
            pltpu.async_copy(dsts_hbm.at[w, j], dst_v.at[d4], isem)

        def wait_idx(s3, d4):
            pltpu.make_async_copy(srcs_hbm.at[w, 0], src_v.at[s3], isem).wait()
            pltpu.make_async_copy(dsts_hbm.at[w, 0], dst_v.at[d4], isem).wait()

        def step(j, p3, pn3, pf3, p4, pn4, pf4, w3, w4,
                 do_swait, do_next, do_pref):
            wait_gather(p3)
            if do_swait:
                wait_scatter(w3, w4)
            issue_scatter(p3, p4)
            if do_next:
                wait_idx(pn3, pn4)
                issue_gather(pn3)
            if do_pref:
                prefetch_idx(j + 2, pf3, pf4)

        pltpu.sync_copy(srcs_hbm.at[w, 0], src_v.at[0])
        pltpu.sync_copy(dsts_hbm.at[w, 0], dst_v.at[0])
        plsc.subcore_barrier()
        issue_gather(0)
        prefetch_idx(1, 1, 1)

        step(0, 0, 1, 2, 0, 1, 2, 0, 0, False, True, True)
        step(1, 1, 2, 0, 1, 2, 3, 0, 0, False, True, True)

        def body(j, carry):
            p3 = lax.rem(j, 3)
            pn3 = lax.rem(j + 1, 3)
            pf3 = lax.rem(j + 2, 3)
            p4 = lax.rem(j, 4)
            pn4 = lax.rem(j + 1, 4)
            pf4 = lax.rem(j + 2, 4)
            step(j, p3, pn3, pf3, p4, pn4, pf4, pn3, pf4, True, True, True)
            return carry

        lax.fori_loop(2, NCH - 2, body, 0)

        jA = NCH - 2
        jB = NCH - 1
        step(jA, jA % 3, jB % 3, 0, jA % 4, jB % 4, 0,
             (jA - 2) % 3, (jA - 2) % 4, True, True, False)
        step(jB, jB % 3, 0, 0, jB % 4, 0, 0,
             (jB - 2) % 3, (jB - 2) % 4, True, False, False)

        wait_scatter(jA % 3, jA % 4)
        wait_scatter(jB % 3, jB % 4)

        plsc.subcore_barrier()
        _write_out(acc, out_hbm, c, s)

    return agg_kernel


_deg_kernel = _make_deg_kernel()
_agg128 = _make_agg_kernel(D_HID)
_agg40 = _make_agg_kernel(D_OUT)

_TCB = 1000


def _tc1_body(deg_ref, x_ref, w_ref, hs_ref, dinv_ref):
    deg = (jnp.sum(deg_ref[...], axis=(0, 2)) * (1.0 / DW))[:, None] + 1.0
    dinv = lax.rsqrt(deg)
    h = jnp.dot(x_ref[...], w_ref[...], preferred_element_type=jnp.float32)
    hs_ref[...] = h * dinv
    dinv_ref[...] = dinv


def _tc1(degp, x, W1):
    grid = (N // _TCB,)
    return pl.pallas_call(
        _tc1_body,
        grid=grid,
        in_specs=[
            pl.BlockSpec((2, _TCB, DW), lambda i: (0, i, 0)),
            pl.BlockSpec((_TCB, D_IN), lambda i: (i, 0)),
            pl.BlockSpec((D_IN, D_HID), lambda i: (0, 0)),
        ],
        out_specs=[
            pl.BlockSpec((_TCB, D_HID), lambda i: (i, 0)),
            pl.BlockSpec((_TCB, 1), lambda i: (i, 0)),
        ],
        out_shape=[
            jax.ShapeDtypeStruct((N, D_HID), jnp.float32),
            jax.ShapeDtypeStruct((N, 1), jnp.float32),
        ],
    )(degp, x, W1)


def _tc2_body(agg_ref, hs1_ref, dinv_ref, b1_ref, w2_ref, hs2_ref):
    dinv = dinv_ref[...]
    o = (agg_ref[0] + agg_ref[1] + hs1_ref[...]) * dinv + b1_ref[...]
    o = jnp.maximum(o, 0.0)
    h2 = jnp.dot(o, w2_ref[...], preferred_element_type=jnp.float32)
    hs2_ref[...] = h2 * dinv


def _tc2(agg, hs1, dinv, b1, W2):
    grid = (N // _TCB,)
    return pl.pallas_call(
        _tc2_body,
        grid=grid,
        in_specs=[
            pl.BlockSpec((2, _TCB, D_HID), lambda i: (0, i, 0)),
            pl.BlockSpec((_TCB, D_HID), lambda i: (i, 0)),
            pl.BlockSpec((_TCB, 1), lambda i: (i, 0)),
            pl.BlockSpec((1, D_HID), lambda i: (0, 0)),
            pl.BlockSpec((D_HID, D_OUT), lambda i: (0, 0)),
        ],
        out_specs=pl.BlockSpec((_TCB, D_OUT), lambda i: (i, 0)),
        out_shape=jax.ShapeDtypeStruct((N, D_OUT), jnp.float32),
    )(agg, hs1, dinv, b1, W2)


def _tc3_body(agg_ref, hs2_ref, dinv_ref, b2_ref, out_ref):
    z = (agg_ref[0] + agg_ref[1] + hs2_ref[...]) * dinv_ref[...] + b2_ref[...]
    m = jnp.max(z, axis=1, keepdims=True)
    e = jnp.exp(z - m)
    lse = jnp.log(jnp.sum(e, axis=1, keepdims=True)) + m
    out_ref[...] = z - lse


def _tc3(agg, hs2, dinv, b2):
    grid = (N // _TCB,)
    return pl.pallas_call(
        _tc3_body,
        grid=grid,
        in_specs=[
            pl.BlockSpec((2, _TCB, D_OUT), lambda i: (0, i, 0)),
            pl.BlockSpec((_TCB, D_OUT), lambda i: (i, 0)),
            pl.BlockSpec((_TCB, 1), lambda i: (i, 0)),
            pl.BlockSpec((1, D_OUT), lambda i: (0, 0)),
        ],
        out_specs=pl.BlockSpec((_TCB, D_OUT), lambda i: (i, 0)),
        out_shape=jax.ShapeDtypeStruct((N, D_OUT), jnp.float32),
    )(agg, hs2, dinv, b2)


def kernel(x, edge_index, W1, b1, W2, b2):
    e = edge_index.astype(jnp.int32)
    src_r = e[0].reshape(NW, NCH, K)
    dst_r = e[1].reshape(NW, NCH, K)

    ones_kd = jnp.ones((K, DW), jnp.float32)
    zeros_d = jnp.zeros((RPW, DW), jnp.float32)
    zeros_h = jnp.zeros((RPW, D_HID), jnp.float32)
    zeros_o = jnp.zeros((RPW, D_OUT), jnp.float32)

    degp = _deg_kernel(dst_r, ones_kd, zeros_d)
    hs1, dinv = _tc1(degp, x, W1)
    agg1 = _agg128(hs1, src_r, dst_r, zeros_h)
    hs2 = _tc2(agg1, hs1, dinv, b1.reshape(1, D_HID), W2)
    agg2 = _agg40(hs2, src_r, dst_r, zeros_o)
    return _tc3(agg2, hs2, dinv, b2.reshape(1, D_OUT))


# speedup vs baseline: 1.4117x; 1.3226x over previous
import functools

import jax
import jax.numpy as jnp
from jax import lax
from jax.experimental import pallas as pl
from jax.experimental.pallas import tpu as pltpu
from jax.experimental.pallas import tpu_sc as plsc

N = 10000
NE = 320000
D_IN = 128
D_HID = 128
D_OUT = 40

NW = 32
EPT = NE // NW
K = 80
NCH = EPT // K

RPW = 640
TAIL = N - 15 * RPW

_mesh = plsc.VectorSubcoreMesh(core_axis_name="c", subcore_axis_name="s")


def _zero_acc(zeros_hbm, acc, s):
    @pl.when(s < 15)
    def _():
        pltpu.sync_copy(zeros_hbm, acc.at[pl.ds(s * RPW, RPW)])

    @pl.when(s == 15)
    def _():
        pltpu.sync_copy(zeros_hbm.at[pl.ds(0, TAIL)], acc.at[pl.ds(15 * RPW, TAIL)])


def _write_out(acc, out_hbm, c, s):
    @pl.when(s < 15)
    def _():
        pltpu.sync_copy(acc.at[pl.ds(s * RPW, RPW)], out_hbm.at[c, pl.ds(s * RPW, RPW)])

    @pl.when(s == 15)
    def _():
        pltpu.sync_copy(acc.at[pl.ds(15 * RPW, TAIL)], out_hbm.at[c, pl.ds(15 * RPW, TAIL)])


DW = 16


def _make_deg_kernel():
    @functools.partial(
        pl.kernel,
        out_type=jax.ShapeDtypeStruct((2, N, DW), jnp.float32),
        mesh=_mesh,
        scratch_types=[
            pltpu.VMEM((4, K), jnp.int32),
            pltpu.VMEM((K, DW), jnp.float32),
            pltpu.VMEM_SHARED((N, DW), jnp.float32),
            pltpu.SemaphoreType.DMA,
            pltpu.SemaphoreType.DMA,
        ],
        compiler_params=pltpu.CompilerParams(use_tc_tiling_on_sc=False),
    )
    def deg_kernel(dsts_hbm, ones_hbm, zeros_hbm, out_hbm,
                   dst_v, ones_v, acc, isem, ssem):
        c = lax.axis_index("c")
        s = lax.axis_index("s")
        w = c * 16 + s
        _zero_acc(zeros_hbm, acc, s)
        pltpu.sync_copy(ones_hbm, ones_v)

        def issue_scatter(p4):
            pltpu.async_copy(ones_v, acc.at[dst_v.at[p4]], ssem, add=True)

        def wait_scatter(p4):
            pltpu.make_async_copy(ones_v, acc.at[dst_v.at[p4]], ssem).wait()

        def prefetch_idx(j, d4):
            pltpu.async_copy(dsts_hbm.at[w, j], dst_v.at[d4], isem)

        def wait_idx(d4):
            pltpu.make_async_copy(dsts_hbm.at[w, 0], dst_v.at[d4], isem).wait()

        pltpu.sync_copy(dsts_hbm.at[w, 0], dst_v.at[0])
        plsc.subcore_barrier()
        issue_scatter(0)
        prefetch_idx(1, 1)
        prefetch_idx(2, 2)
        wait_idx(1)
        issue_scatter(1)
        prefetch_idx(3, 3)

        def body(j, carry):
            p4 = lax.rem(j, 4)
            pf4 = lax.rem(j + 2, 4)
            w4 = lax.rem(j + 2, 4)
            wait_scatter(w4)
            wait_idx(p4)
            issue_scatter(p4)
            prefetch_idx(j + 2, pf4)
            return carry

        lax.fori_loop(2, NCH - 2, body, 0)

        jA = NCH - 2
        jB = NCH - 1
        wait_scatter((jA - 2) % 4)
        wait_idx(jA % 4)
        issue_scatter(jA % 4)
        wait_scatter((jB - 2) % 4)
        wait_idx(jB % 4)
        issue_scatter(jB % 4)
        wait_scatter(jA % 4)
        wait_scatter(jB % 4)

        plsc.subcore_barrier()
        _write_out(acc, out_hbm, c, s)

    return deg_kernel


def _make_agg_kernel(D):
    @functools.partial(
        pl.kernel,
        out_type=jax.ShapeDtypeStruct((2, N, D), jnp.float32),
        mesh=_mesh,
        scratch_types=[
            pltpu.VMEM((4, K), jnp.int32),
            pltpu.VMEM((8, K), jnp.int32),
            pltpu.VMEM((4, K, D), jnp.float32),
            pltpu.VMEM_SHARED((N, D), jnp.float32),
            pltpu.SemaphoreType.DMA,
            pltpu.SemaphoreType.DMA,
            pltpu.SemaphoreType.DMA,
            pltpu.SemaphoreType.DMA,
            pltpu.SemaphoreType.DMA,
        ],
        compiler_params=pltpu.CompilerParams(use_tc_tiling_on_sc=False),
    )
    def agg_kernel(hs_hbm, srcs_hbm, dsts_hbm, zeros_hbm, out_hbm,
                   src_v, dst_v, stage, acc, gsE, gsO, isem, ssE, ssO):
        c = lax.axis_index("c")
        s = lax.axis_index("s")
        w = c * 16 + s
        _zero_acc(zeros_hbm, acc, s)

        def gather(c4, sem):
            pltpu.async_copy(hs_hbm.at[src_v.at[c4]], stage.at[c4], sem)

        def gwait(c4, sem):
            pltpu.make_async_copy(hs_hbm.at[src_v.at[c4]], stage.at[c4], sem).wait()

        def scat(c4, c8, sem):
            pltpu.async_copy(stage.at[c4], acc.at[dst_v.at[c8]], sem, add=True)

        def swait(c4, c8, sem):
            pltpu.make_async_copy(stage.at[c4], acc.at[dst_v.at[c8]], sem).wait()

        def pref(j, c4, c8):
            pltpu.async_copy(srcs_hbm.at[w, j], src_v.at[c4], isem)
            pltpu.async_copy(dsts_hbm.at[w, j], dst_v.at[c8], isem)

        def pwait(c4, c8):
            pltpu.make_async_copy(srcs_hbm.at[w, 0], src_v.at[c4], isem).wait()
            pltpu.make_async_copy(dsts_hbm.at[w, 0], dst_v.at[c8], isem).wait()

        def step(j, j4, j8, g4, g8, gsem, ssem, w4, w8,
                 do_swait, do_gather, pj, p4, p8):
            gwait(j4, gsem)
            if do_swait:
                swait(w4, w8, ssem)
            scat(j4, j8, ssem)
            if do_gather:
                pwait(g4, g8)
                gather(g4, gsem)
            if pj is not None:
                pref(pj, p4, p8)

        pltpu.sync_copy(srcs_hbm.at[w, 0], src_v.at[0])
        pltpu.sync_copy(dsts_hbm.at[w, 0], dst_v.at[0])
        plsc.subcore_barrier()
        gather(0, gsE)
        pref(1, 1, 1)
        pwait(1, 1)
        gather(1, gsO)
        pref(2, 2, 2)

        step(0, 0, 0, 2, 2, gsE, ssE, 0, 0, False, True, 3, 3, 3)
        step(1, 1, 1, 3, 3, gsO, ssO, 0, 0, False, True, 4, 0, 4)

        def body(t, carry):
            j = 2 * t
            step(j, lax.rem(j, 4), lax.rem(j, 8),
                 lax.rem(j + 2, 4), lax.rem(j + 2, 8), gsE, ssE,
                 lax.rem(j + 2, 4), lax.rem(j + 6, 8),
                 True, True, j + 3, lax.rem(j + 3, 4), lax.rem(j + 3, 8))
            step(j + 1, lax.rem(j + 1, 4), lax.rem(j + 1, 8),
                 lax.rem(j + 3, 4), lax.rem(j + 3, 8), gsO, ssO,
                 lax.rem(j + 3, 4), lax.rem(j + 7, 8),
                 True, True, j + 4, lax.rem(j + 4, 4), lax.rem(j + 4, 8))
            return carry

        lax.fori_loop(1, (NCH - 3) // 2, body, 0)

        jA = NCH - 3
        jB = NCH - 2
        jC = NCH - 1
        step(jA, jA % 4, jA % 8, jC % 4, jC % 8, gsE, ssE,
             (jA - 2) % 4, (jA - 2) % 8, True, True, None, 0, 0)
        step(jB, jB % 4, jB % 8, 0, 0, gsO, ssO,
             (jB - 2) % 4, (jB - 2) % 8, True, False, None, 0, 0)
        step(jC, jC % 4, jC % 8, 0, 0, gsE, ssE,
             (jC - 2) % 4, (jC - 2) % 8, True, False, None, 0, 0)

        swait(jB % 4, jB % 8, ssO)
        swait(jC % 4, jC % 8, ssE)

        plsc.subcore_barrier()
        _write_out(acc, out_hbm, c, s)

    return agg_kernel


_deg_kernel = _make_deg_kernel()
_agg128 = _make_agg_kernel(D_HID)
_agg40 = _make_agg_kernel(D_OUT)

_TCB = 1000


def _tc1_body(deg_ref, x_ref, w_ref, hs_ref, dinv_ref):
    deg = (jnp.sum(deg_ref[...], axis=(0, 2)) * (1.0 / DW))[:, None] + 1.0
    dinv = lax.rsqrt(deg)
    h = jnp.dot(x_ref[...], w_ref[...], preferred_element_type=jnp.float32)
    hs_ref[...] = h * dinv
    dinv_ref[...] = dinv


def _tc1(degp, x, W1):
    grid = (N // _TCB,)
    return pl.pallas_call(
        _tc1_body,
        grid=grid,
        in_specs=[
            pl.BlockSpec((2, _TCB, DW), lambda i: (0, i, 0)),
            pl.BlockSpec((_TCB, D_IN), lambda i: (i, 0)),
            pl.BlockSpec((D_IN, D_HID), lambda i: (0, 0)),
        ],
        out_specs=[
            pl.BlockSpec((_TCB, D_HID), lambda i: (i, 0)),
            pl.BlockSpec((_TCB, 1), lambda i: (i, 0)),
        ],
        out_shape=[
            jax.ShapeDtypeStruct((N, D_HID), jnp.float32),
            jax.ShapeDtypeStruct((N, 1), jnp.float32),
        ],
    )(degp, x, W1)


def _tc2_body(agg_ref, hs1_ref, dinv_ref, b1_ref, w2_ref, hs2_ref):
    dinv = dinv_ref[...]
    o = (agg_ref[0] + agg_ref[1] + hs1_ref[...]) * dinv + b1_ref[...]
    o = jnp.maximum(o, 0.0)
    h2 = jnp.dot(o, w2_ref[...], preferred_element_type=jnp.float32)
    hs2_ref[...] = h2 * dinv


def _tc2(agg, hs1, dinv, b1, W2):
    grid = (N // _TCB,)
    return pl.pallas_call(
        _tc2_body,
        grid=grid,
        in_specs=[
            pl.BlockSpec((2, _TCB, D_HID), lambda i: (0, i, 0)),
            pl.BlockSpec((_TCB, D_HID), lambda i: (i, 0)),
            pl.BlockSpec((_TCB, 1), lambda i: (i, 0)),
            pl.BlockSpec((1, D_HID), lambda i: (0, 0)),
            pl.BlockSpec((D_HID, D_OUT), lambda i: (0, 0)),
        ],
        out_specs=pl.BlockSpec((_TCB, D_OUT), lambda i: (i, 0)),
        out_shape=jax.ShapeDtypeStruct((N, D_OUT), jnp.float32),
    )(agg, hs1, dinv, b1, W2)


def _tc3_body(agg_ref, hs2_ref, dinv_ref, b2_ref, out_ref):
    z = (agg_ref[0] + agg_ref[1] + hs2_ref[...]) * dinv_ref[...] + b2_ref[...]
    m = jnp.max(z, axis=1, keepdims=True)
    e = jnp.exp(z - m)
    lse = jnp.log(jnp.sum(e, axis=1, keepdims=True)) + m
    out_ref[...] = z - lse


def _tc3(agg, hs2, dinv, b2):
    grid = (N // _TCB,)
    return pl.pallas_call(
        _tc3_body,
        grid=grid,
        in_specs=[
            pl.BlockSpec((2, _TCB, D_OUT), lambda i: (0, i, 0)),
            pl.BlockSpec((_TCB, D_OUT), lambda i: (i, 0)),
            pl.BlockSpec((_TCB, 1), lambda i: (i, 0)),
            pl.BlockSpec((1, D_OUT), lambda i: (0, 0)),
        ],
        out_specs=pl.BlockSpec((_TCB, D_OUT), lambda i: (i, 0)),
        out_shape=jax.ShapeDtypeStruct((N, D_OUT), jnp.float32),
    )(agg, hs2, dinv, b2)


def kernel(x, edge_index, W1, b1, W2, b2):
    e = edge_index.astype(jnp.int32)
    src_r = e[0].reshape(NW, NCH, K)
    dst_r = e[1].reshape(NW, NCH, K)

    ones_kd = jnp.ones((K, DW), jnp.float32)
    zeros_d = jnp.zeros((RPW, DW), jnp.float32)
    zeros_h = jnp.zeros((RPW, D_HID), jnp.float32)
    zeros_o = jnp.zeros((RPW, D_OUT), jnp.float32)

    degp = _deg_kernel(dst_r, ones_kd, zeros_d)
    hs1, dinv = _tc1(degp, x, W1)
    agg1 = _agg128(hs1, src_r, dst_r, zeros_h)
    hs2 = _tc2(agg1, hs1, dinv, b1.reshape(1, D_HID), W2)
    agg2 = _agg40(hs2, src_r, dst_r, zeros_o)
    return _tc3(agg2, hs2, dinv, b2.reshape(1, D_OUT))
